# baseline (device time: 38057 ns/iter reference)
import jax
import jax.numpy as jnp
from jax import lax
from jax.experimental import pallas as pl
from jax.experimental.pallas import tpu as pltpu

N_DEV = 4
N_LAYERS = 3
NS = 4
N_PHASES = 1 + N_LAYERS * NS
N_SLOTS = 2 * NS


def kernel(x, Win0, Wout0, Win1, Wout1, Win2, Wout2):
    b, d = x.shape
    B = N_DEV * b
    sb = B // NS

    def body(x_ref, win0, wout0, win1, wout1, win2, wout2,
             out_ref, xg, pbuf, rbuf, wv0, wv1, wv2, wv3, wv4, wv5,
             dma_sems, send_sems, recv_sems):
        my = lax.axis_index("i")

        w_hbm = (win0, wout0, win1, wout1, win2, wout2)
        wvmem = (wv0, wv1, wv2, wv3, wv4, wv5)
        w_dmas = []
        for i, w in enumerate(w_hbm):
            dma = pltpu.make_async_copy(w, wvmem[i], dma_sems.at[i])
            dma.start()
            w_dmas.append(dma)

        barrier = pltpu.get_barrier_semaphore()
        for o in (1, 2, 3):
            pl.semaphore_signal(
                barrier, inc=1,
                device_id=((my + o) % N_DEV,),
                device_id_type=pl.DeviceIdType.MESH,
            )
        pl.semaphore_wait(barrier, N_DEV - 1)

        def ar_descriptor(phase, q, o):
            dest = (my + o) % N_DEV
            return pltpu.make_async_remote_copy(
                src_ref=pbuf.at[pl.ds(q * sb, sb)],
                dst_ref=rbuf.at[(phase - 1) % N_SLOTS,
                                (my - dest) % N_DEV - 1],
                send_sem=send_sems.at[phase, o - 1],
                recv_sem=recv_sems.at[phase, (my - dest) % N_DEV - 1],
                device_id=(dest,),
                device_id_type=pl.DeviceIdType.MESH,
            )

        def recv_wait(phase, o):
            dest = (my + o) % N_DEV
            pltpu.make_async_remote_copy(
                src_ref=pbuf.at[pl.ds(0, sb)],
                dst_ref=rbuf.at[(phase - 1) % N_SLOTS, o - 1],
                send_sem=send_sems.at[phase, o - 1],
                recv_sem=recv_sems.at[phase, o - 1],
                device_id=(dest,),
                device_id_type=pl.DeviceIdType.MESH,
            ).wait_recv()

        def reduced(phase, rows):
            s = (phase - 1) % N_SLOTS
            return (
                (pbuf[rows, :] + rbuf[s, 0, :, :])
                + (rbuf[s, 1, :, :] + rbuf[s, 2, :, :])
            )

        my_rows = pl.ds(my * b, b)
        xg[my_rows, :] = x_ref[:, :].astype(jnp.bfloat16)
        ag_rdmas = []
        for o in (1, 2, 3):
            dest = (my + o) % N_DEV
            rdma = pltpu.make_async_remote_copy(
                src_ref=xg.at[my_rows],
                dst_ref=xg.at[my_rows],
                send_sem=send_sems.at[0, o - 1],
                recv_sem=recv_sems.at[0, (my - dest) % N_DEV - 1],
                device_id=(dest,),
                device_id_type=pl.DeviceIdType.MESH,
            )
            rdma.start()
            ag_rdmas.append(rdma)

        def ag_wait(o):
            pltpu.make_async_remote_copy(
                src_ref=xg.at[my_rows],
                dst_ref=xg.at[pl.ds(((my + o) % N_DEV) * b, b)],
                send_sem=send_sems.at[0, o - 1],
                recv_sem=recv_sems.at[0, o - 1],
                device_id=((my + o) % N_DEV,),
                device_id_type=pl.DeviceIdType.MESH,
            ).wait_recv()

        prev_sends = {q: [] for q in range(NS)}
        for k in range(N_LAYERS):
            w_dmas[2 * k].wait()
            win_b = wvmem[2 * k][:, :].astype(jnp.bfloat16)
            w_dmas[2 * k + 1].wait()
            wout_b = wvmem[2 * k + 1][:, :].astype(jnp.bfloat16)
            for q in range(NS):
                rows = pl.ds(q * sb, sb)
                phase = 1 + NS * k + q
                if k > 0:
                    pphase = phase - NS
                    for o in (1, 2, 3):
                        recv_wait(pphase, o)
                    xg[rows, :] = reduced(pphase, rows)
                else:
                    for o in (1, 2, 3):
                        pl.when((my + o) % N_DEV == q)(
                            lambda o=o: ag_wait(o))
                h = jnp.dot(xg[rows, :], win_b,
                            preferred_element_type=jnp.float32)
                h = jnp.maximum(h, 0.0).astype(jnp.bfloat16)
                for rdma in prev_sends[q]:
                    rdma.wait_send()
                pbuf[rows, :] = jnp.dot(
                    h, wout_b, preferred_element_type=jnp.float32
                ).astype(jnp.bfloat16)
                sends = []
                for o in (1, 2, 3):
                    rdma = ar_descriptor(phase, q, o)
                    rdma.start()
                    sends.append(rdma)
                prev_sends[q] = sends
            if k == 0:
                for rdma in ag_rdmas:
                    rdma.wait_send()

        for q in range(NS):
            rows = pl.ds(q * sb, sb)
            phase = 1 + NS * (N_LAYERS - 1) + q
            for o in (1, 2, 3):
                recv_wait(phase, o)
            out_ref[rows, :] = reduced(phase, rows).astype(jnp.float32)
        for q in range(NS):
            for rdma in prev_sends[q]:
                rdma.wait_send()

    return pl.pallas_call(
        body,
        out_shape=jax.ShapeDtypeStruct((B, d), jnp.float32),
        in_specs=[pl.BlockSpec(memory_space=pltpu.VMEM)]
        + [pl.BlockSpec(memory_space=pl.ANY)] * 6,
        out_specs=pl.BlockSpec(memory_space=pltpu.VMEM),
        scratch_shapes=[
            pltpu.VMEM((B, d), jnp.bfloat16),
            pltpu.VMEM((B, d), jnp.bfloat16),
            pltpu.VMEM((N_SLOTS, 3, B // NS, d), jnp.bfloat16),
            pltpu.VMEM(Win0.shape, jnp.float32),
            pltpu.VMEM(Wout0.shape, jnp.float32),
            pltpu.VMEM(Win0.shape, jnp.float32),
            pltpu.VMEM(Wout0.shape, jnp.float32),
            pltpu.VMEM(Win0.shape, jnp.float32),
            pltpu.VMEM(Wout0.shape, jnp.float32),
            pltpu.SemaphoreType.DMA((6,)),
            pltpu.SemaphoreType.DMA((N_PHASES, 3)),
            pltpu.SemaphoreType.DMA((N_PHASES, 3)),
        ],
        compiler_params=pltpu.CompilerParams(collective_id=0),
    )(x, Win0, Wout0, Win1, Wout1, Win2, Wout2)


# device time: 36055 ns/iter; 1.0555x vs baseline; 1.0555x over previous
import jax
import jax.numpy as jnp
from jax import lax
from jax.experimental import pallas as pl
from jax.experimental.pallas import tpu as pltpu

N_DEV = 4
N_LAYERS = 3
NS = 4
N_PHASES = 1 + N_LAYERS * NS
N_SLOTS = 2 * NS


def kernel(x, Win0, Wout0, Win1, Wout1, Win2, Wout2):
    b, d = x.shape
    B = N_DEV * b
    sb = B // NS

    def body(x_ref, win0, wout0, win1, wout1, win2, wout2,
             out_ref, xg, pbuf, rbuf, send_sems, recv_sems):
        my = lax.axis_index("i")

        barrier = pltpu.get_barrier_semaphore()
        for o in (1, 2, 3):
            pl.semaphore_signal(
                barrier, inc=1,
                device_id=((my + o) % N_DEV,),
                device_id_type=pl.DeviceIdType.MESH,
            )
        pl.semaphore_wait(barrier, N_DEV - 1)

        def ar_descriptor(phase, q, o):
            dest = (my + o) % N_DEV
            return pltpu.make_async_remote_copy(
                src_ref=pbuf.at[pl.ds(q * sb, sb)],
                dst_ref=rbuf.at[(phase - 1) % N_SLOTS,
                                (my - dest) % N_DEV - 1],
                send_sem=send_sems.at[phase, o - 1],
                recv_sem=recv_sems.at[phase, (my - dest) % N_DEV - 1],
                device_id=(dest,),
                device_id_type=pl.DeviceIdType.MESH,
            )

        def recv_wait(phase, o):
            dest = (my + o) % N_DEV
            pltpu.make_async_remote_copy(
                src_ref=pbuf.at[pl.ds(0, sb)],
                dst_ref=rbuf.at[(phase - 1) % N_SLOTS, o - 1],
                send_sem=send_sems.at[phase, o - 1],
                recv_sem=recv_sems.at[phase, o - 1],
                device_id=(dest,),
                device_id_type=pl.DeviceIdType.MESH,
            ).wait_recv()

        def reduced(phase, rows):
            s = (phase - 1) % N_SLOTS
            return (
                (pbuf[rows, :] + rbuf[s, 0, :, :])
                + (rbuf[s, 1, :, :] + rbuf[s, 2, :, :])
            )

        my_rows = pl.ds(my * b, b)
        xg[my_rows, :] = x_ref[:, :]
        ag_rdmas = []
        for o in (1, 2, 3):
            dest = (my + o) % N_DEV
            rdma = pltpu.make_async_remote_copy(
                src_ref=xg.at[my_rows],
                dst_ref=xg.at[my_rows],
                send_sem=send_sems.at[0, o - 1],
                recv_sem=recv_sems.at[0, (my - dest) % N_DEV - 1],
                device_id=(dest,),
                device_id_type=pl.DeviceIdType.MESH,
            )
            rdma.start()
            ag_rdmas.append(rdma)

        def ag_wait(o):
            pltpu.make_async_remote_copy(
                src_ref=xg.at[my_rows],
                dst_ref=xg.at[pl.ds(((my + o) % N_DEV) * b, b)],
                send_sem=send_sems.at[0, o - 1],
                recv_sem=recv_sems.at[0, o - 1],
                device_id=((my + o) % N_DEV,),
                device_id_type=pl.DeviceIdType.MESH,
            ).wait_recv()

        weights = ((win0, wout0), (win1, wout1), (win2, wout2))
        prev_sends = {q: [] for q in range(NS)}
        for k, (win, wout) in enumerate(weights):
            win_b = win[:, :]
            wout_b = wout[:, :]
            for q in range(NS):
                rows = pl.ds(q * sb, sb)
                phase = 1 + NS * k + q
                if k > 0:
                    pphase = phase - NS
                    for o in (1, 2, 3):
                        recv_wait(pphase, o)
                    xg[rows, :] = reduced(pphase, rows)
                else:
                    for o in (1, 2, 3):
                        pl.when((my + o) % N_DEV == q)(
                            lambda o=o: ag_wait(o))
                h = jnp.dot(xg[rows, :], win_b,
                            preferred_element_type=jnp.float32)
                h = jnp.maximum(h, 0.0).astype(jnp.bfloat16)
                for rdma in prev_sends[q]:
                    rdma.wait_send()
                pbuf[rows, :] = jnp.dot(
                    h, wout_b, preferred_element_type=jnp.float32
                ).astype(jnp.bfloat16)
                sends = []
                for o in (1, 2, 3):
                    rdma = ar_descriptor(phase, q, o)
                    rdma.start()
                    sends.append(rdma)
                prev_sends[q] = sends
            if k == 0:
                for rdma in ag_rdmas:
                    rdma.wait_send()

        for q in range(NS):
            rows = pl.ds(q * sb, sb)
            phase = 1 + NS * (N_LAYERS - 1) + q
            for o in (1, 2, 3):
                recv_wait(phase, o)
            out_ref[rows, :] = reduced(phase, rows).astype(jnp.float32)
        for q in range(NS):
            for rdma in prev_sends[q]:
                rdma.wait_send()

    return pl.pallas_call(
        body,
        out_shape=jax.ShapeDtypeStruct((B, d), jnp.float32),
        in_specs=[pl.BlockSpec(memory_space=pltpu.VMEM)] * 7,
        out_specs=pl.BlockSpec(memory_space=pltpu.VMEM),
        scratch_shapes=[
            pltpu.VMEM((B, d), jnp.bfloat16),
            pltpu.VMEM((B, d), jnp.bfloat16),
            pltpu.VMEM((N_SLOTS, 3, B // NS, d), jnp.bfloat16),
            pltpu.SemaphoreType.DMA((N_PHASES, 3)),
            pltpu.SemaphoreType.DMA((N_PHASES, 3)),
        ],
        compiler_params=pltpu.CompilerParams(collective_id=0),
    )(x.astype(jnp.bfloat16),
      Win0.astype(jnp.bfloat16), Wout0.astype(jnp.bfloat16),
      Win1.astype(jnp.bfloat16), Wout1.astype(jnp.bfloat16),
      Win2.astype(jnp.bfloat16), Wout2.astype(jnp.bfloat16))


# device time: 35954 ns/iter; 1.0585x vs baseline; 1.0028x over previous
import jax
import jax.numpy as jnp
from jax import lax
from jax.experimental import pallas as pl
from jax.experimental.pallas import tpu as pltpu

N_DEV = 4
N_LAYERS = 3
NS = 4
N_PHASES = 1 + N_LAYERS * NS
N_SLOTS = 2 * NS


def kernel(x, Win0, Wout0, Win1, Wout1, Win2, Wout2):
    b, d = x.shape
    B = N_DEV * b
    sb = B // NS

    def body(x_ref, win0, wout0, win1, wout1, win2, wout2,
             out_ref, xg, pbuf, rbuf, send_sems, recv_sems):
        my = lax.axis_index("i")

        barrier = pltpu.get_barrier_semaphore()
        for o in (1, 2, 3):
            pl.semaphore_signal(
                barrier, inc=1,
                device_id=((my + o) % N_DEV,),
                device_id_type=pl.DeviceIdType.MESH,
            )
        pl.semaphore_wait(barrier, N_DEV - 1)

        def ar_descriptor(phase, q, o):
            dest = (my + o) % N_DEV
            return pltpu.make_async_remote_copy(
                src_ref=pbuf.at[pl.ds(q * sb, sb)],
                dst_ref=rbuf.at[(phase - 1) % N_SLOTS,
                                (my - dest) % N_DEV - 1],
                send_sem=send_sems.at[phase, o - 1],
                recv_sem=recv_sems.at[phase, (my - dest) % N_DEV - 1],
                device_id=(dest,),
                device_id_type=pl.DeviceIdType.MESH,
            )

        def recv_wait(phase, o):
            dest = (my + o) % N_DEV
            pltpu.make_async_remote_copy(
                src_ref=pbuf.at[pl.ds(0, sb)],
                dst_ref=rbuf.at[(phase - 1) % N_SLOTS, o - 1],
                send_sem=send_sems.at[phase, o - 1],
                recv_sem=recv_sems.at[phase, o - 1],
                device_id=(dest,),
                device_id_type=pl.DeviceIdType.MESH,
            ).wait_recv()

        def reduced(phase, rows):
            s = (phase - 1) % N_SLOTS
            return (
                (pbuf[rows, :] + rbuf[s, 0, :, :])
                + (rbuf[s, 1, :, :] + rbuf[s, 2, :, :])
            )

        my_rows = pl.ds(my * b, b)
        xg[my_rows, :] = x_ref[:, :]
        ag_rdmas = []
        for o in (1, 2, 3):
            dest = (my + o) % N_DEV
            rdma = pltpu.make_async_remote_copy(
                src_ref=xg.at[my_rows],
                dst_ref=xg.at[my_rows],
                send_sem=send_sems.at[0, o - 1],
                recv_sem=recv_sems.at[0, (my - dest) % N_DEV - 1],
                device_id=(dest,),
                device_id_type=pl.DeviceIdType.MESH,
            )
            rdma.start()
            ag_rdmas.append(rdma)

        def ag_wait(o):
            pltpu.make_async_remote_copy(
                src_ref=xg.at[my_rows],
                dst_ref=xg.at[pl.ds(((my + o) % N_DEV) * b, b)],
                send_sem=send_sems.at[0, o - 1],
                recv_sem=recv_sems.at[0, o - 1],
                device_id=((my + o) % N_DEV,),
                device_id_type=pl.DeviceIdType.MESH,
            ).wait_recv()

        weights = ((win0, wout0), (win1, wout1), (win2, wout2))
        prev_sends = {q: [] for q in range(NS)}
        for k, (win, wout) in enumerate(weights):
            win_b = win[:, :]
            wout_b = wout[:, :]
            for q in range(NS):
                rows = pl.ds(q * sb, sb)
                phase = 1 + NS * k + q
                if k > 0:
                    pphase = phase - NS
                    for o in (1, 2, 3):
                        recv_wait(pphase, o)
                    xg[rows, :] = reduced(pphase, rows)
                else:
                    for o in (1, 2, 3):
                        pl.when((my + o) % N_DEV == q)(
                            lambda o=o: ag_wait(o))
                h = jnp.dot(xg[rows, :], win_b,
                            preferred_element_type=jnp.float32)
                h = jnp.maximum(h, 0.0).astype(jnp.bfloat16)
                for rdma in prev_sends[q]:
                    rdma.wait_send()
                pbuf[rows, :] = jnp.dot(
                    h, wout_b, preferred_element_type=jnp.float32
                ).astype(jnp.bfloat16)
                sends = []
                for o in (1, 2, 3):
                    rdma = ar_descriptor(phase, q, o)
                    rdma.start()
                    sends.append(rdma)
                prev_sends[q] = sends
            if k == 0:
                for rdma in ag_rdmas:
                    rdma.wait_send()

        for q in range(NS):
            rows = pl.ds(q * sb, sb)
            phase = 1 + NS * (N_LAYERS - 1) + q
            for o in (1, 2, 3):
                recv_wait(phase, o)
            out_ref[rows, :] = reduced(phase, rows)
        for q in range(NS):
            for rdma in prev_sends[q]:
                rdma.wait_send()

    return pl.pallas_call(
        body,
        out_shape=jax.ShapeDtypeStruct((B, d), jnp.bfloat16),
        in_specs=[pl.BlockSpec(memory_space=pltpu.VMEM)] * 7,
        out_specs=pl.BlockSpec(memory_space=pltpu.VMEM),
        scratch_shapes=[
            pltpu.VMEM((B, d), jnp.bfloat16),
            pltpu.VMEM((B, d), jnp.bfloat16),
            pltpu.VMEM((N_SLOTS, 3, B // NS, d), jnp.bfloat16),
            pltpu.SemaphoreType.DMA((N_PHASES, 3)),
            pltpu.SemaphoreType.DMA((N_PHASES, 3)),
        ],
        compiler_params=pltpu.CompilerParams(collective_id=0),
    )(x.astype(jnp.bfloat16),
      Win0.astype(jnp.bfloat16), Wout0.astype(jnp.bfloat16),
      Win1.astype(jnp.bfloat16), Wout1.astype(jnp.bfloat16),
      Win2.astype(jnp.bfloat16), Wout2.astype(jnp.bfloat16))
